# baseline (speedup 1.0000x reference)
import jax
import jax.numpy as jnp
from jax import lax
from jax.experimental import pallas as pl
from jax.experimental.pallas import tpu as pltpu
from jax.experimental.pallas import tpu_sc as plsc
from jax._src.pallas import mpmd as _mpmd

INPUT_SIZE = 64
HIDDEN = 64
W = 128
CAST = 512
N_STORIES = 1024
M = N_STORIES * CAST
B = 16384
N_STOP = 64

NC = 2
NS = 16
NW = NC * NS
BPW = B // NW
CHUNK = 128
NCHUNK = BPW // CHUNK

_mesh = plsc.VectorSubcoreMesh(
    core_axis_name="c", subcore_axis_name="s", num_cores=NC, num_subcores=NS
)
_sc_params = pltpu.CompilerParams(use_tc_tiling_on_sc=False)


def _wid():
  return lax.axis_index("s") * NC + lax.axis_index("c")


def _sc_gather_body(state_hbm, idx_hbm, out_hbm, tbl_out, idx_v, rows_v, sem):
  del tbl_out
  wid = _wid()
  pltpu.sync_copy(idx_hbm.at[pl.ds(wid * NCHUNK, NCHUNK)], idx_v)
  descs = []
  for j in range(NCHUNK):
    descs.append(
        pltpu.async_copy(
            state_hbm.at[idx_v.at[j]],
            rows_v.at[pl.ds(j * CHUNK, CHUNK)],
            sem,
        )
    )
  for d in descs:
    d.wait()
  pltpu.sync_copy(rows_v, out_hbm.at[pl.ds(wid * BPW, BPW)])


_sc_gather = _mpmd._mpmd_map(
    [(_mesh, _sc_gather_body)],
    out_types=(
        jax.ShapeDtypeStruct((B, W), jnp.float32),
        jax.ShapeDtypeStruct((M, W), jnp.float32),
    ),
    input_output_aliases={0: 1},
    scratch_types=[
        pltpu.VMEM((NCHUNK, CHUNK), jnp.int32),
        pltpu.VMEM((BPW, W), jnp.float32),
        pltpu.SemaphoreType.DMA,
    ],
    compiler_params=_sc_params,
)


def _sc_scatter_body(tbl_in, sidx_hbm, ord_hbm, rows_hbm, out_hbm,
                     sidx_v, ord_v, rows_v, sem):
  del tbl_in
  wid = _wid()
  pltpu.sync_copy(sidx_hbm.at[pl.ds(wid * NCHUNK, NCHUNK)], sidx_v)
  pltpu.sync_copy(ord_hbm.at[pl.ds(wid * NCHUNK, NCHUNK)], ord_v)
  descs = []
  for j in range(NCHUNK):
    descs.append(
        pltpu.async_copy(
            rows_hbm.at[ord_v.at[j]],
            rows_v.at[pl.ds(j * CHUNK, CHUNK)],
            sem,
        )
    )
  for d in descs:
    d.wait()
  descs = []
  for j in range(NCHUNK):
    descs.append(
        pltpu.async_copy(
            rows_v.at[pl.ds(j * CHUNK, CHUNK)],
            out_hbm.at[sidx_v.at[j]],
            sem,
        )
    )
  for d in descs:
    d.wait()


_sc_scatter = _mpmd._mpmd_map(
    [(_mesh, _sc_scatter_body)],
    out_types=jax.ShapeDtypeStruct((M, W), jnp.float32),
    input_output_aliases={0: 0},
    scratch_types=[
        pltpu.VMEM((NCHUNK, CHUNK), jnp.int32),
        pltpu.VMEM((NCHUNK, CHUNK), jnp.int32),
        pltpu.VMEM((BPW, W), jnp.float32),
        pltpu.SemaphoreType.DMA,
    ],
    compiler_params=_sc_params,
)


def _sc_zero_body(tbl_in, stop_hbm, zeros_hbm, out_hbm, stop_v, zeros_v):
  del tbl_in
  @pl.when(_wid() == 0)
  def _():
    pltpu.sync_copy(stop_hbm, stop_v)
    pltpu.sync_copy(zeros_hbm, zeros_v)
    pltpu.sync_copy(zeros_v, out_hbm.at[stop_v])


_sc_zero = _mpmd._mpmd_map(
    [(_mesh, _sc_zero_body)],
    out_types=jax.ShapeDtypeStruct((M, W), jnp.float32),
    input_output_aliases={0: 0},
    scratch_types=[
        pltpu.VMEM((N_STOP,), jnp.int32),
        pltpu.VMEM((N_STOP, W), jnp.float32),
    ],
    compiler_params=_sc_params,
)


_GRU_BS = 2048


def _gru_body(x_ref, h_ref, wih_ref, whh_ref, bih_ref, bhh_ref, out_ref):
  x = x_ref[...]
  h = h_ref[:, :HIDDEN]
  dn = (((1,), (1,)), ((), ()))
  gi = lax.dot_general(x, wih_ref[...], dn,
                       preferred_element_type=jnp.float32) + bih_ref[...]
  gh = lax.dot_general(h, whh_ref[...], dn,
                       preferred_element_type=jnp.float32) + bhh_ref[...]
  i_r, i_z, i_n = gi[:, :HIDDEN], gi[:, HIDDEN:2 * HIDDEN], gi[:, 2 * HIDDEN:]
  h_r, h_z, h_n = gh[:, :HIDDEN], gh[:, HIDDEN:2 * HIDDEN], gh[:, 2 * HIDDEN:]
  r = jax.nn.sigmoid(i_r + h_r)
  z = jax.nn.sigmoid(i_z + h_z)
  n = jnp.tanh(i_n + r * h_n)
  hnew = (1.0 - z) * n + z * h
  out_ref[...] = jnp.concatenate(
      [hnew, jnp.zeros((_GRU_BS, W - HIDDEN), jnp.float32)], axis=1)


_gru = pl.pallas_call(
    _gru_body,
    grid=(B // _GRU_BS,),
    in_specs=[
        pl.BlockSpec((_GRU_BS, INPUT_SIZE), lambda i: (i, 0)),
        pl.BlockSpec((_GRU_BS, W), lambda i: (i, 0)),
        pl.BlockSpec((3 * HIDDEN, INPUT_SIZE), lambda i: (0, 0)),
        pl.BlockSpec((3 * HIDDEN, HIDDEN), lambda i: (0, 0)),
        pl.BlockSpec((1, 3 * HIDDEN), lambda i: (0, 0)),
        pl.BlockSpec((1, 3 * HIDDEN), lambda i: (0, 0)),
    ],
    out_specs=pl.BlockSpec((_GRU_BS, W), lambda i: (i, 0)),
    out_shape=jax.ShapeDtypeStruct((B, W), jnp.float32),
)


_WID_BS = 4096


def _widen_body(s_ref, eye_ref, o_ref):
  blk = s_ref[...]
  left = lax.dot_general(blk, eye_ref[...], (((0,), (0,)), ((), ())),
                         precision=lax.Precision.HIGHEST,
                         preferred_element_type=jnp.float32)
  o_ref[:, :HIDDEN] = left


_widen = pl.pallas_call(
    _widen_body,
    grid=(M // _WID_BS,),
    in_specs=[
        pl.BlockSpec((HIDDEN, _WID_BS), lambda i: (0, i)),
        pl.BlockSpec((HIDDEN, HIDDEN), lambda i: (0, 0)),
    ],
    out_specs=pl.BlockSpec((_WID_BS, W), lambda i: (i, 0)),
    out_shape=jax.ShapeDtypeStruct((M, W), jnp.float32),
)


def kernel(x, state, batch_idxs, actor_ids, story_stop_idxs, W_ih, W_hh,
           b_ih, b_hh):
  aid = jnp.clip(actor_ids, 0, CAST - 1).astype(jnp.int32)
  idxs = batch_idxs.astype(jnp.int32) * CAST + aid

  pos = jnp.arange(B, dtype=jnp.int32)
  sidx, order = lax.sort((idxs, pos), dimension=0, is_stable=True, num_keys=1)
  keep_s = jnp.concatenate(
      [sidx[:-1] != sidx[1:], jnp.ones((1,), jnp.bool_)])
  stop0 = story_stop_idxs[0].astype(jnp.int32)
  scat_idx = jnp.where(keep_s, sidx, stop0)

  table0 = _widen(state.T, jnp.eye(HIDDEN, dtype=jnp.float32))

  selected, table = _sc_gather(table0, idxs.reshape(B // CHUNK, CHUNK))
  new_sel_wide = _gru(x, selected, W_ih, W_hh,
                      b_ih.reshape(1, 3 * HIDDEN), b_hh.reshape(1, 3 * HIDDEN))
  scattered = _sc_scatter(table, scat_idx.reshape(B // CHUNK, CHUNK),
                          order.reshape(B // CHUNK, CHUNK), new_sel_wide)
  final_wide = _sc_zero(scattered, story_stop_idxs.astype(jnp.int32),
                        jnp.zeros((N_STOP, W), jnp.float32))
  new_selected = new_sel_wide[:, :HIDDEN]
  new_state = final_wide[:, :HIDDEN]
  return new_selected, new_state


# speedup vs baseline: 1.2023x; 1.2023x over previous
import jax
import jax.numpy as jnp
from jax import lax
from jax.experimental import pallas as pl
from jax.experimental.pallas import tpu as pltpu
from jax.experimental.pallas import tpu_sc as plsc
from jax._src.pallas import mpmd as _mpmd

INPUT_SIZE = 64
HIDDEN = 64
W = 128
CAST = 512
N_STORIES = 1024
M = N_STORIES * CAST
B = 16384
N_STOP = 64

NC = 2
NS = 16
NW = NC * NS
BPW = B // NW
CHUNK = 128
NCHUNK = BPW // CHUNK

_mesh = plsc.VectorSubcoreMesh(
    core_axis_name="c", subcore_axis_name="s", num_cores=NC, num_subcores=NS
)
_sc_params = pltpu.CompilerParams(use_tc_tiling_on_sc=False)


def _wid():
  return lax.axis_index("s") * NC + lax.axis_index("c")


def _sc_gather_body(state_hbm, idx_hbm, out_hbm, tbl_out, idx_v, rows_v, sem):
  del tbl_out
  wid = _wid()
  pltpu.sync_copy(idx_hbm.at[pl.ds(wid * NCHUNK, NCHUNK)], idx_v)
  descs = []
  for j in range(NCHUNK):
    descs.append(
        pltpu.async_copy(
            state_hbm.at[idx_v.at[j]],
            rows_v.at[pl.ds(j * CHUNK, CHUNK)],
            sem,
        )
    )
  for d in descs:
    d.wait()
  pltpu.sync_copy(rows_v, out_hbm.at[pl.ds(wid * BPW, BPW)])


_sc_gather = _mpmd._mpmd_map(
    [(_mesh, _sc_gather_body)],
    out_types=(
        jax.ShapeDtypeStruct((B, W), jnp.float32),
        jax.ShapeDtypeStruct((M, W), jnp.float32),
    ),
    input_output_aliases={0: 1},
    scratch_types=[
        pltpu.VMEM((NCHUNK, CHUNK), jnp.int32),
        pltpu.VMEM((BPW, W), jnp.float32),
        pltpu.SemaphoreType.DMA,
    ],
    compiler_params=_sc_params,
)


def _sc_scatter_body(tbl_in, sidx_hbm, ord_hbm, rows_hbm, out_hbm,
                     sidx_v, ord_v, rows_v, sem):
  del tbl_in
  wid = _wid()
  pltpu.sync_copy(sidx_hbm.at[pl.ds(wid * NCHUNK, NCHUNK)], sidx_v)
  pltpu.sync_copy(ord_hbm.at[pl.ds(wid * NCHUNK, NCHUNK)], ord_v)
  descs = []
  for j in range(NCHUNK):
    descs.append(
        pltpu.async_copy(
            rows_hbm.at[ord_v.at[j]],
            rows_v.at[pl.ds(j * CHUNK, CHUNK)],
            sem,
        )
    )
  for d in descs:
    d.wait()
  descs = []
  for j in range(NCHUNK):
    descs.append(
        pltpu.async_copy(
            rows_v.at[pl.ds(j * CHUNK, CHUNK)],
            out_hbm.at[sidx_v.at[j]],
            sem,
        )
    )
  for d in descs:
    d.wait()


_sc_scatter = _mpmd._mpmd_map(
    [(_mesh, _sc_scatter_body)],
    out_types=jax.ShapeDtypeStruct((M, W), jnp.float32),
    input_output_aliases={0: 0},
    scratch_types=[
        pltpu.VMEM((NCHUNK, CHUNK), jnp.int32),
        pltpu.VMEM((NCHUNK, CHUNK), jnp.int32),
        pltpu.VMEM((BPW, W), jnp.float32),
        pltpu.SemaphoreType.DMA,
    ],
    compiler_params=_sc_params,
)


def _sc_zero_body(tbl_in, stop_hbm, zeros_hbm, out_hbm, stop_v, zeros_v):
  del tbl_in
  @pl.when(_wid() == 0)
  def _():
    pltpu.sync_copy(stop_hbm, stop_v)
    pltpu.sync_copy(zeros_hbm, zeros_v)
    pltpu.sync_copy(zeros_v, out_hbm.at[stop_v])


_sc_zero = _mpmd._mpmd_map(
    [(_mesh, _sc_zero_body)],
    out_types=jax.ShapeDtypeStruct((M, W), jnp.float32),
    input_output_aliases={0: 0},
    scratch_types=[
        pltpu.VMEM((N_STOP,), jnp.int32),
        pltpu.VMEM((N_STOP, W), jnp.float32),
    ],
    compiler_params=_sc_params,
)


_GRU_BS = 2048


def _gru_body(x_ref, h_ref, wih_ref, whh_ref, bih_ref, bhh_ref, out_ref):
  x = x_ref[...]
  h = h_ref[:, :HIDDEN]
  dn = (((1,), (1,)), ((), ()))
  gi = lax.dot_general(x, wih_ref[...], dn,
                       preferred_element_type=jnp.float32) + bih_ref[...]
  gh = lax.dot_general(h, whh_ref[...], dn,
                       preferred_element_type=jnp.float32) + bhh_ref[...]
  i_r, i_z, i_n = gi[:, :HIDDEN], gi[:, HIDDEN:2 * HIDDEN], gi[:, 2 * HIDDEN:]
  h_r, h_z, h_n = gh[:, :HIDDEN], gh[:, HIDDEN:2 * HIDDEN], gh[:, 2 * HIDDEN:]
  r = jax.nn.sigmoid(i_r + h_r)
  z = jax.nn.sigmoid(i_z + h_z)
  n = jnp.tanh(i_n + r * h_n)
  hnew = (1.0 - z) * n + z * h
  out_ref[...] = jnp.concatenate(
      [hnew, jnp.zeros((_GRU_BS, W - HIDDEN), jnp.float32)], axis=1)


_gru = pl.pallas_call(
    _gru_body,
    grid=(B // _GRU_BS,),
    in_specs=[
        pl.BlockSpec((_GRU_BS, INPUT_SIZE), lambda i: (i, 0)),
        pl.BlockSpec((_GRU_BS, W), lambda i: (i, 0)),
        pl.BlockSpec((3 * HIDDEN, INPUT_SIZE), lambda i: (0, 0)),
        pl.BlockSpec((3 * HIDDEN, HIDDEN), lambda i: (0, 0)),
        pl.BlockSpec((1, 3 * HIDDEN), lambda i: (0, 0)),
        pl.BlockSpec((1, 3 * HIDDEN), lambda i: (0, 0)),
    ],
    out_specs=pl.BlockSpec((_GRU_BS, W), lambda i: (i, 0)),
    out_shape=jax.ShapeDtypeStruct((B, W), jnp.float32),
)


_WID_BS = 4096


def _widen_body(s_ref, eye_ref, o_ref):
  blk = s_ref[...]
  left = blk.T
  o_ref[:, :HIDDEN] = left


_widen = pl.pallas_call(
    _widen_body,
    grid=(M // _WID_BS,),
    in_specs=[
        pl.BlockSpec((HIDDEN, _WID_BS), lambda i: (0, i)),
        pl.BlockSpec((HIDDEN, HIDDEN), lambda i: (0, 0)),
    ],
    out_specs=pl.BlockSpec((_WID_BS, W), lambda i: (i, 0)),
    out_shape=jax.ShapeDtypeStruct((M, W), jnp.float32),
)


def kernel(x, state, batch_idxs, actor_ids, story_stop_idxs, W_ih, W_hh,
           b_ih, b_hh):
  aid = jnp.clip(actor_ids, 0, CAST - 1).astype(jnp.int32)
  idxs = batch_idxs.astype(jnp.int32) * CAST + aid

  pos = jnp.arange(B, dtype=jnp.int32)
  sidx, order = lax.sort((idxs, pos), dimension=0, is_stable=True, num_keys=1)
  keep_s = jnp.concatenate(
      [sidx[:-1] != sidx[1:], jnp.ones((1,), jnp.bool_)])
  stop0 = story_stop_idxs[0].astype(jnp.int32)
  scat_idx = jnp.where(keep_s, sidx, stop0)

  table0 = _widen(state.T, jnp.eye(HIDDEN, dtype=jnp.float32))

  selected, table = _sc_gather(table0, idxs.reshape(B // CHUNK, CHUNK))
  new_sel_wide = _gru(x, selected, W_ih, W_hh,
                      b_ih.reshape(1, 3 * HIDDEN), b_hh.reshape(1, 3 * HIDDEN))
  scattered = _sc_scatter(table, scat_idx.reshape(B // CHUNK, CHUNK),
                          order.reshape(B // CHUNK, CHUNK), new_sel_wide)
  final_wide = _sc_zero(scattered, story_stop_idxs.astype(jnp.int32),
                        jnp.zeros((N_STOP, W), jnp.float32))
  new_selected = new_sel_wide[:, :HIDDEN]
  new_state = final_wide[:, :HIDDEN]
  return new_selected, new_state
